# baseline (speedup 1.0000x reference)
import functools

import jax
import jax.numpy as jnp
from jax import lax
from jax.experimental import pallas as pl
from jax.experimental.pallas import tpu as pltpu
from jax.experimental.pallas import tpu_sc as plsc

_N_ATT = 8
_M = 128
_C = 768
_HW = 224 * 224
_BLK = 512
_LANES = 16
_CHUNK = _HW // _LANES
_NBINS = 32
_NPASS = 7
_NROWS = 2 * _N_ATT


def _map_desc(u):
    t = jnp.right_shift(u, 31)
    mask = jnp.bitwise_and(jnp.bitwise_not(t), jnp.int32(0x7FFFFFFF))
    return jnp.bitwise_xor(u, mask)


def _hash_body(x_ref, rmt_ref, out_ref):
    x = x_ref[...].astype(jnp.bfloat16)
    w = rmt_ref[...].astype(jnp.bfloat16)
    rot = lax.dot_general(w, x, (((1,), (0,)), ((), ())),
                          preferred_element_type=jnp.float32)
    h = jnp.max(rot.reshape(_N_ATT, _M, _BLK), axis=1)
    u = lax.bitcast_convert_type(h, jnp.int32)
    out_ref[...] = _map_desc(u)


def _hash_call(x, rmt):
    n = x.shape[0]
    return pl.pallas_call(
        _hash_body,
        grid=(n, _HW // _BLK),
        in_specs=[
            pl.BlockSpec((None, _C, _BLK), lambda i, j: (i, 0, j)),
            pl.BlockSpec((_N_ATT * _M, _C), lambda i, j: (0, 0)),
        ],
        out_specs=pl.BlockSpec((None, _N_ATT, _BLK), lambda i, j: (i, 0, j)),
        out_shape=jax.ShapeDtypeStruct((n, _N_ATT, _HW), jnp.int32),
    )(x, rmt)


def _sort_body(keys_hbm, out_hbm, buf_a, buf_b, hist):
    cid = lax.axis_index("c")
    sid = lax.axis_index("s")
    wid = sid * 2 + cid

    @pl.when(wid < _NROWS)
    def _work():
        row = wid
        pltpu.sync_copy(keys_hbm.at[row], buf_a)
        lane = lax.broadcasted_iota(jnp.int32, (_LANES,), 0)
        ones = jnp.ones((_LANES,), jnp.int32)
        zeros = jnp.zeros((_LANES,), jnp.int32)
        thirty_one = jnp.full((_LANES,), 31, jnp.int32)

        for p in range(_NPASS):
            src = buf_a if p % 2 == 0 else buf_b
            dst = buf_b if p % 2 == 0 else buf_a
            sh = jnp.full((_LANES,), 5 * p, jnp.int32)
            last = p == _NPASS - 1

            for b in range(_NBINS):
                hist[b] = zeros

            def body_a(i, carry, src=src, sh=sh):
                idx = lane * _CHUNK + i
                k = plsc.load_gather(src, [idx])
                d = jnp.bitwise_and(lax.shift_right_logical(k, sh), thirty_one)
                plsc.addupdate_scatter(hist, [d, lane], ones)
                return carry

            lax.fori_loop(0, _CHUNK, body_a, 0)

            g = jnp.int32(0)
            for b in range(_NBINS):
                rowv = hist[b]
                incl = plsc.cumsum(rowv)
                hist[b] = incl - rowv + g
                g = g + jnp.sum(rowv)

            def body_c(i, carry, src=src, dst=dst, sh=sh, last=last):
                idx = lane * _CHUNK + i
                k = plsc.load_gather(src, [idx])
                d = jnp.bitwise_and(lax.shift_right_logical(k, sh), thirty_one)
                pos = plsc.load_gather(hist, [d, lane])
                kout = _map_desc(k) if last else k
                plsc.store_scatter(dst, [pos], kout)
                plsc.addupdate_scatter(hist, [d, lane], ones)
                return carry

            lax.fori_loop(0, _CHUNK, body_c, 0)

        pltpu.sync_copy(buf_b, out_hbm.at[row])


@functools.partial(jax.jit, static_argnums=())
def _sort_call(keys):
    mesh = plsc.VectorSubcoreMesh(core_axis_name="c", subcore_axis_name="s")
    fn = pl.kernel(
        _sort_body,
        out_type=jax.ShapeDtypeStruct((_NROWS, _HW), jnp.int32),
        mesh=mesh,
        compiler_params=pltpu.CompilerParams(needs_layout_passes=False),
        scratch_types=[
            pltpu.VMEM((_HW,), jnp.int32),
            pltpu.VMEM((_HW,), jnp.int32),
            pltpu.VMEM((_NBINS, _LANES), jnp.int32),
        ],
    )
    return fn(keys)


def kernel(inputs, random_matrices):
    n, c, h, w = inputs.shape
    x = inputs.reshape(n, c, h * w)
    rmt = random_matrices.transpose(0, 2, 1).reshape(_N_ATT * _M, _C)
    keys = _hash_call(x, rmt)
    sorted_keys = _sort_call(keys.reshape(_NROWS, _HW))
    out = lax.bitcast_convert_type(sorted_keys, jnp.float32)
    return out.reshape(n, _N_ATT, h * w)


# speedup vs baseline: 1.6640x; 1.6640x over previous
import functools

import jax
import jax.numpy as jnp
from jax import lax
from jax.experimental import pallas as pl
from jax.experimental.pallas import tpu as pltpu
from jax.experimental.pallas import tpu_sc as plsc

_N_ATT = 8
_M = 128
_C = 768
_HW = 224 * 224
_BLK = 1024
_LANES = 16
_CH = _HW // _LANES
_CHP = _CH + 1
_PADN = _LANES * _CHP
_NBINS = 256
_NPASS = 4
_NROWS = 2 * _N_ATT
_UNROLL = 8


def _map_desc(u):
    t = jnp.right_shift(u, 31)
    mask = jnp.bitwise_and(jnp.bitwise_not(t), jnp.int32(0x7FFFFFFF))
    return jnp.bitwise_xor(u, mask)


def _hash_body(x_ref, rmt_ref, out_ref):
    x = x_ref[...].astype(jnp.bfloat16)
    w = rmt_ref[...]
    rot = lax.dot_general(w, x, (((1,), (0,)), ((), ())),
                          preferred_element_type=jnp.float32)
    h = jnp.max(rot.reshape(_N_ATT, _M, _BLK), axis=1)
    u = lax.bitcast_convert_type(h, jnp.int32)
    out_ref[...] = _map_desc(u)


def _hash_call(x, rmt_bf16):
    n = x.shape[0]
    return pl.pallas_call(
        _hash_body,
        grid=(n, _HW // _BLK),
        in_specs=[
            pl.BlockSpec((None, _C, _BLK), lambda i, j: (i, 0, j)),
            pl.BlockSpec((_N_ATT * _M, _C), lambda i, j: (0, 0)),
        ],
        out_specs=pl.BlockSpec((None, _N_ATT, _BLK), lambda i, j: (i, 0, j)),
        out_shape=jax.ShapeDtypeStruct((n, _N_ATT, _HW), jnp.int32),
    )(x, rmt_bf16)


def _radix_pass(src, dst, hist, p):
    last = p == _NPASS - 1
    lane = lax.broadcasted_iota(jnp.int32, (_LANES,), 0)
    ones = jnp.ones((_LANES,), jnp.int32)
    zeros = jnp.zeros((_LANES,), jnp.int32)
    sh = jnp.full((_LANES,), 8 * p, jnp.int32)
    m255 = jnp.full((_LANES,), 255, jnp.int32)
    inv_ch = jnp.full((_LANES,), 1.0 / _CH, jnp.float32)
    half = jnp.full((_LANES,), 0.5, jnp.float32)
    lane_chp = lane * _CHP

    def read(i):
        if p == 0:
            return src[pl.ds(i * _LANES, _LANES)]
        return plsc.load_gather(src, [lane_chp + i])

    def digit(k):
        return jnp.bitwise_and(lax.shift_right_logical(k, sh), m255)

    def body_zero(b, carry):
        plsc.store_scatter(hist, [b * _LANES + lane], zeros)
        return carry

    lax.fori_loop(0, _NBINS, body_zero, 0)

    def body_a(j, carry):
        for u in range(_UNROLL):
            i = j * _UNROLL + u
            plsc.addupdate_scatter(hist, [digit(read(i)) * _LANES + lane], ones)
        return carry

    lax.fori_loop(0, _CH // _UNROLL, body_a, 0)

    def body_b(b, g):
        addr = b * _LANES + lane
        vec = plsc.load_gather(hist, [addr])
        incl = plsc.cumsum(vec)
        plsc.store_scatter(hist, [addr], incl - vec + g)
        return g + jnp.sum(vec)

    lax.fori_loop(0, _NBINS, body_b, jnp.int32(0))

    def body_c(j, carry):
        for u in range(_UNROLL):
            i = j * _UNROLL + u
            k = read(i)
            addr = digit(k) * _LANES + lane
            pos = plsc.load_gather(hist, [addr])
            plsc.addupdate_scatter(hist, [addr], ones)
            if last:
                plsc.store_scatter(dst, [pos], _map_desc(k))
            else:
                q = ((pos.astype(jnp.float32) + half) * inv_ch).astype(jnp.int32)
                plsc.store_scatter(dst, [pos + q], k)
        return carry

    lax.fori_loop(0, _CH // _UNROLL, body_c, 0)


def _sort_body(keys_hbm, out_hbm, buf_a, buf_b, hist):
    cid = lax.axis_index("c")
    sid = lax.axis_index("s")
    wid = sid * 2 + cid

    @pl.when(wid < _NROWS)
    def _work():
        row = wid
        pltpu.sync_copy(keys_hbm.at[row], buf_a.at[pl.ds(0, _HW)])
        _radix_pass(buf_a, buf_b, hist, 0)
        _radix_pass(buf_b, buf_a, hist, 1)
        _radix_pass(buf_a, buf_b, hist, 2)
        _radix_pass(buf_b, buf_a, hist, 3)
        pltpu.sync_copy(buf_a.at[pl.ds(0, _HW)], out_hbm.at[row])


def _sort_call(keys):
    mesh = plsc.VectorSubcoreMesh(core_axis_name="c", subcore_axis_name="s")
    fn = pl.kernel(
        _sort_body,
        out_type=jax.ShapeDtypeStruct((_NROWS, _HW), jnp.int32),
        mesh=mesh,
        compiler_params=pltpu.CompilerParams(needs_layout_passes=False),
        scratch_types=[
            pltpu.VMEM((_PADN,), jnp.int32),
            pltpu.VMEM((_PADN,), jnp.int32),
            pltpu.VMEM((_NBINS * _LANES,), jnp.int32),
        ],
    )
    return fn(keys)


def kernel(inputs, random_matrices):
    n, c, h, w = inputs.shape
    x = inputs.reshape(n, c, h * w)
    rmt = random_matrices.transpose(0, 2, 1).reshape(_N_ATT * _M, _C)
    keys = _hash_call(x, rmt.astype(jnp.bfloat16))
    sorted_keys = _sort_call(keys.reshape(_NROWS, _HW))
    out = lax.bitcast_convert_type(sorted_keys, jnp.float32)
    return out.reshape(n, _N_ATT, h * w)
